# Initial kernel scaffold; baseline (speedup 1.0000x reference)
import jax
import jax.numpy as jnp
from jax.experimental import pallas as pl


def kernel(atom_types, pos, edge_index, batch, emb, rbf_offsets, lin1_W, filt_W1, filt_b1, filt_W2, filt_b2, lin2_W, lin2_b, lin_W, lin_b, out_W1, out_b1, out_W2, out_b2):
    raise NotImplementedError("write your pallas kernel here")


# speedup vs baseline: 2.3566x; 2.3566x over previous
import functools
import math

import jax
import jax.numpy as jnp
from jax import lax
from jax.experimental import pallas as pl
from jax.experimental.pallas import tpu as pltpu
from jax.experimental.pallas import tpu_sc as plsc

N_NODES = 10000
N_EDGES = 320000
HIDDEN = 128
NUM_RBF = 50
N_BLOCKS = 3
N_MOL = 32
CUTOFF = 5.0

NC = 2
NS = 16
NW = NC * NS
CH = 128
NCHUNK = 80
EPW = CH * NCHUNK
E_PAD = EPW * NW
NPAD = 10240
RPW = NPAD // NS
RBF_PAD = 64
EC = 4096
LANES = 16


def _mesh():
    return plsc.VectorSubcoreMesh(core_axis_name="c", subcore_axis_name="s")


_SC_PARAMS = pltpu.CompilerParams(needs_layout_passes=False)


def _d2_body(px_hbm, py_hbm, pz_hbm, src_hbm, dst_hbm, out_hbm,
             px_v, py_v, pz_v, si_v, di_v, d2_v):
    c = lax.axis_index("c")
    s = lax.axis_index("s")
    w = s * NC + c
    base = w * EPW
    pltpu.sync_copy(px_hbm, px_v)
    pltpu.sync_copy(py_hbm, py_v)
    pltpu.sync_copy(pz_hbm, pz_v)
    pltpu.sync_copy(src_hbm.at[pl.ds(base, EPW)], si_v)
    pltpu.sync_copy(dst_hbm.at[pl.ds(base, EPW)], di_v)

    def step(it, carry):
        sl = pl.ds(it * LANES, LANES)
        si = si_v[sl]
        di = di_v[sl]
        dx = plsc.load_gather(px_v, [di]) - plsc.load_gather(px_v, [si])
        dy = plsc.load_gather(py_v, [di]) - plsc.load_gather(py_v, [si])
        dz = plsc.load_gather(pz_v, [di]) - plsc.load_gather(pz_v, [si])
        d2_v[sl] = dx * dx + dy * dy + dz * dz
        return carry

    lax.fori_loop(0, EPW // LANES, step, 0)
    pltpu.sync_copy(d2_v, out_hbm.at[pl.ds(base, EPW)])


_d2_kernel = pl.kernel(
    _d2_body,
    out_type=jax.ShapeDtypeStruct((E_PAD,), jnp.float32),
    mesh=_mesh(),
    compiler_params=_SC_PARAMS,
    scratch_types=[
        pltpu.VMEM((N_NODES,), jnp.float32),
        pltpu.VMEM((N_NODES,), jnp.float32),
        pltpu.VMEM((N_NODES,), jnp.float32),
        pltpu.VMEM((EPW,), jnp.int32),
        pltpu.VMEM((EPW,), jnp.int32),
        pltpu.VMEM((EPW,), jnp.float32),
    ],
)


def _make_sc_block(i):
    def body(h_hbm, wf_hbm, srcm_hbm, dstm_hbm, zeros_hbm, out_hbm,
             si_v, di_v, h_buf, wf_buf, agg_sh, isem, lsem, gsem):
        c = lax.axis_index("c")
        s = lax.axis_index("s")
        w = s * NC + c
        pltpu.sync_copy(zeros_hbm.at[pl.ds(s * RPW, RPW)],
                        agg_sh.at[pl.ds(s * RPW, RPW)])
        pltpu.sync_copy(dstm_hbm.at[w], di_v)
        plsc.subcore_barrier()

        def chunk(j, carry):
            base = w * EPW + j * CH
            si_cp = pltpu.async_copy(srcm_hbm.at[w, j], si_v, isem)
            l = pltpu.async_copy(wf_hbm.at[i, pl.ds(base, CH)], wf_buf, lsem)
            si_cp.wait()
            g = pltpu.async_copy(h_hbm.at[si_v], h_buf, gsem)
            l.wait()
            g.wait()

            def row(k, carry2):
                for t in range(HIDDEN // LANES):
                    sl = pl.ds(t * LANES, LANES)
                    h_buf[k, sl] = h_buf[k, sl] * wf_buf[k, sl]
                return carry2

            lax.fori_loop(0, CH, row, 0)
            pltpu.sync_copy(h_buf, agg_sh.at[di_v.at[j]], add=True)
            return carry

        lax.fori_loop(0, NCHUNK, chunk, 0)
        plsc.subcore_barrier()
        pltpu.sync_copy(agg_sh.at[pl.ds(s * RPW, RPW)],
                        out_hbm.at[c, pl.ds(s * RPW, RPW)])

    return pl.kernel(
        body,
        out_type=jax.ShapeDtypeStruct((NC, NPAD, HIDDEN), jnp.float32),
        mesh=_mesh(),
        compiler_params=_SC_PARAMS,
        scratch_types=[
            pltpu.VMEM((CH,), jnp.int32),
            pltpu.VMEM((NCHUNK, CH), jnp.int32),
            pltpu.VMEM((CH, HIDDEN), jnp.float32),
            pltpu.VMEM((CH, HIDDEN), jnp.float32),
            pltpu.VMEM_SHARED((NPAD, HIDDEN), jnp.float32),
            pltpu.SemaphoreType.DMA,
            pltpu.SemaphoreType.DMA,
            pltpu.SemaphoreType.DMA,
        ],
    )


_sc_block = [_make_sc_block(i) for i in range(N_BLOCKS)]


def _emb_body(t_ref, emb_ref, w_ref, x_ref, h_ref):
    oh = (t_ref[...] == lax.broadcasted_iota(jnp.int32, (NPAD, HIDDEN), 1))
    x = jnp.dot(oh.astype(jnp.float32), emb_ref[...],
                preferred_element_type=jnp.float32)
    x_ref[...] = x
    h_ref[...] = jnp.dot(x, w_ref[...], preferred_element_type=jnp.float32)


_emb_call = pl.pallas_call(
    _emb_body,
    out_shape=(jax.ShapeDtypeStruct((NPAD, HIDDEN), jnp.float32),
               jax.ShapeDtypeStruct((NPAD, HIDDEN), jnp.float32)),
)


def _wf_body(d2_ref, offs_ref, w1_ref, b1_ref, w2_ref, b2_ref, out_ref):
    d = jnp.sqrt(d2_ref[...] + 1e-12)
    step = offs_ref[0, 1] - offs_ref[0, 0]
    coeff = -0.5 / (step * step)
    offs = offs_ref[0:1, :]
    rbf = jnp.exp(coeff * (d - offs) ** 2)
    e0 = pl.program_id(0) * EC
    eid = e0 + lax.broadcasted_iota(jnp.int32, (EC, 1), 0)
    valid = (eid < N_EDGES).astype(jnp.float32)
    C = 0.5 * (jnp.cos(d * (math.pi / CUTOFF)) + 1.0)
    C = C * (d < CUTOFF).astype(jnp.float32) * valid
    for i in range(N_BLOCKS):
        t = jnp.tanh(jnp.dot(rbf, w1_ref[i], preferred_element_type=jnp.float32)
                     + b1_ref[i])
        wf = (jnp.dot(t, w2_ref[i], preferred_element_type=jnp.float32)
              + b2_ref[i]) * C
        out_ref[i] = wf


_wf_call = pl.pallas_call(
    _wf_body,
    grid=(E_PAD // EC,),
    in_specs=[
        pl.BlockSpec((EC, 1), lambda e: (e, 0)),
        pl.BlockSpec((8, RBF_PAD), lambda e: (0, 0)),
        pl.BlockSpec((N_BLOCKS, RBF_PAD, HIDDEN), lambda e: (0, 0, 0)),
        pl.BlockSpec((N_BLOCKS, 1, HIDDEN), lambda e: (0, 0, 0)),
        pl.BlockSpec((N_BLOCKS, HIDDEN, HIDDEN), lambda e: (0, 0, 0)),
        pl.BlockSpec((N_BLOCKS, 1, HIDDEN), lambda e: (0, 0, 0)),
    ],
    out_specs=pl.BlockSpec((N_BLOCKS, EC, HIDDEN), lambda e: (0, e, 0)),
    out_shape=jax.ShapeDtypeStruct((N_BLOCKS, E_PAD, HIDDEN), jnp.float32),
)


def _upd_body(x_ref, agg_ref, w2l_ref, b2l_ref, wl_ref, bl_ref, w1n_ref,
              xo_ref, ho_ref):
    agg = agg_ref[0] + agg_ref[1]
    a = jnp.dot(agg, w2l_ref[...], preferred_element_type=jnp.float32) + b2l_ref[...]
    u = jnp.tanh(a)
    xn = x_ref[...] + jnp.dot(u, wl_ref[...], preferred_element_type=jnp.float32) + bl_ref[...]
    xo_ref[...] = xn
    ho_ref[...] = jnp.dot(xn, w1n_ref[...], preferred_element_type=jnp.float32)


_upd_call = pl.pallas_call(
    _upd_body,
    out_shape=(jax.ShapeDtypeStruct((NPAD, HIDDEN), jnp.float32),
               jax.ShapeDtypeStruct((NPAD, HIDDEN), jnp.float32)),
)


def _fin_body(x_ref, agg_ref, w2l_ref, b2l_ref, wl_ref, bl_ref,
              ow1_ref, ob1_ref, ow2t_ref, ob2_ref, batch_ref, eo_ref):
    agg = agg_ref[0] + agg_ref[1]
    u = jnp.tanh(jnp.dot(agg, w2l_ref[...], preferred_element_type=jnp.float32)
                 + b2l_ref[...])
    xn = x_ref[...] + jnp.dot(u, wl_ref[...], preferred_element_type=jnp.float32) + bl_ref[...]
    hout = jnp.tanh(jnp.dot(xn, ow1_ref[...], preferred_element_type=jnp.float32)
                    + ob1_ref[...])
    e = jnp.sum(hout * ow2t_ref[...], axis=1, keepdims=True) + ob2_ref[0, 0]
    oh = (batch_ref[...] == lax.broadcasted_iota(jnp.int32, (NPAD, N_MOL), 1))
    energy = jnp.sum(e * oh.astype(jnp.float32), axis=0, keepdims=True)
    eo_ref[...] = jnp.broadcast_to(energy, (8, N_MOL))


_fin_call = pl.pallas_call(
    _fin_body,
    out_shape=jax.ShapeDtypeStruct((8, N_MOL), jnp.float32),
)


def kernel(atom_types, pos, edge_index, batch, emb, rbf_offsets,
           lin1_W, filt_W1, filt_b1, filt_W2, filt_b2,
           lin2_W, lin2_b, lin_W, lin_b,
           out_W1, out_b1, out_W2, out_b2):
    f32 = jnp.float32
    i32 = jnp.int32
    src = edge_index[0].astype(i32)
    dst = edge_index[1].astype(i32)
    pad = E_PAD - N_EDGES
    zpad = jnp.zeros((pad,), i32)
    src_p = jnp.concatenate([src, zpad])
    dst_p = jnp.concatenate([dst, zpad])
    srcm = src_p.reshape(NW, NCHUNK, CH)
    dstm = dst_p.reshape(NW, NCHUNK, CH)

    pos32 = pos.astype(f32)
    d2 = _d2_kernel(pos32[:, 0], pos32[:, 1], pos32[:, 2], src_p, dst_p)

    npad = NPAD - N_NODES
    types2 = jnp.concatenate([atom_types.astype(i32),
                              jnp.zeros((npad,), i32)]).reshape(NPAD, 1)
    emb_p = jnp.zeros((HIDDEN, HIDDEN), f32).at[:emb.shape[0]].set(emb.astype(f32))
    x, h = _emb_call(types2, emb_p, lin1_W[0].astype(f32))

    offs_p = jnp.full((RBF_PAD,), 1e15, f32).at[:NUM_RBF].set(rbf_offsets.astype(f32))
    offs_p = jnp.broadcast_to(offs_p[None, :], (8, RBF_PAD))
    w1_p = jnp.zeros((N_BLOCKS, RBF_PAD, HIDDEN), f32)
    w1_p = w1_p.at[:, :NUM_RBF].set(filt_W1.astype(f32))
    wfall = _wf_call(d2.reshape(E_PAD, 1), offs_p, w1_p,
                     filt_b1.reshape(N_BLOCKS, 1, HIDDEN).astype(f32),
                     filt_W2.astype(f32),
                     filt_b2.reshape(N_BLOCKS, 1, HIDDEN).astype(f32))

    zeros = jnp.zeros((NPAD, HIDDEN), f32)
    batch2 = jnp.concatenate([batch.astype(i32),
                              jnp.full((npad,), N_MOL, i32)]).reshape(NPAD, 1)
    for i in range(N_BLOCKS):
        aggp = _sc_block[i](h, wfall, srcm, dstm, zeros)
        if i + 1 < N_BLOCKS:
            x, h = _upd_call(x, aggp,
                             lin2_W[i].astype(f32), lin2_b[i].reshape(1, HIDDEN).astype(f32),
                             lin_W[i].astype(f32), lin_b[i].reshape(1, HIDDEN).astype(f32),
                             lin1_W[i + 1].astype(f32))
        else:
            e8 = _fin_call(x, aggp,
                           lin2_W[i].astype(f32), lin2_b[i].reshape(1, HIDDEN).astype(f32),
                           lin_W[i].astype(f32), lin_b[i].reshape(1, HIDDEN).astype(f32),
                           out_W1.astype(f32), out_b1.reshape(1, HIDDEN // 2).astype(f32),
                           out_W2.reshape(1, HIDDEN // 2).astype(f32),
                           out_b2.reshape(1, 1).astype(f32),
                           batch2)
    return e8[0]
